# baseline (speedup 1.0000x reference)
import jax
import jax.numpy as jnp
from jax.experimental import pallas as pl
from jax.experimental.pallas import tpu as pltpu

_N = 5
_K = 5
_C = 64
_HW = 25
_HW2 = 32
_MS = _K * _HW
_MSP = 640
_MU = 2500
_MUP = 2560
_QT = 15
_NQT = 5
_QR = _QT * _HW2
_INVSQ = 0.125
_TEMP = 2.0
_NEG = -jnp.inf
_VW = _N * _C + _N


def _dot(a, b, dims):
    return jax.lax.dot_general(a, b, (dims, ((), ())),
                               preferred_element_type=jnp.float32)


def _select_body(sup_ref, unl_ref, wv_ref,
                 svt_ref, vall_ref, vunl_ref, cnt_ref):
    supf = sup_ref[0].reshape(_N * _MSP, _C)
    unl = unl_ref[0]
    wv = wv_ref[...]
    cn = ((1,), (1,))
    bias = jnp.where(
        jax.lax.broadcasted_iota(jnp.int32, (_N * _MSP, 1), 0) % _MSP
        < _MS, 0.0, -1e30)
    svt_ref[0] = _dot(supf, wv, cn).reshape(_N, _MSP, _C)
    uvt = _dot(unl, wv, cn)

    sn = supf / jnp.maximum(
        jnp.sqrt(jnp.sum(supf * supf, axis=1, keepdims=True)), 1e-12)
    sn3 = jnp.concatenate([sn, bias], axis=1).reshape(_N, _MSP, _C + 1)
    un = unl / jnp.maximum(
        jnp.sqrt(jnp.sum(unl * unl, axis=1, keepdims=True)), 1e-12)
    un = jnp.concatenate([un, jnp.ones((_MUP, 1), jnp.float32)], axis=1)

    ch = 640
    nch = _MUP // ch
    rms, umax5s = [], []
    colmax = [jnp.full((1, _MSP), _NEG, jnp.float32) for _ in range(_N)]
    for ci in range(nch):
        uc = un[ci * ch:(ci + 1) * ch]
        if ci == nch - 1:
            rowval = (jax.lax.broadcasted_iota(jnp.int32, (ch, 1), 0)
                      < _MU - ci * ch)
        else:
            rowval = None
        um5 = []
        for n in range(_N):
            cc = _dot(uc, sn3[n], cn)
            um5.append(jnp.max(cc, axis=1, keepdims=True))
            ccr = cc if rowval is None else jnp.where(rowval, cc, _NEG)
            colmax[n] = jnp.maximum(colmax[n],
                                    jnp.max(ccr, axis=0, keepdims=True))
        umax5 = jnp.concatenate(um5, axis=1)
        rms.append(jnp.max(umax5, axis=1, keepdims=True))
        umax5s.append(umax5)

    mnns = []
    for ci in range(nch):
        uc = un[ci * ch:(ci + 1) * ch]
        if ci == nch - 1:
            rowval = (jax.lax.broadcasted_iota(jnp.int32, (ch, 1), 0)
                      < _MU - ci * ch).astype(jnp.float32)
        else:
            rowval = None
        mn = jnp.zeros((ch, 1), jnp.float32)
        for n in range(_N):
            cc = _dot(uc, sn3[n], cn)
            hit = jnp.where((cc >= rms[ci]) & (cc >= colmax[n]), 1.0, 0.0)
            mn = jnp.maximum(mn, jnp.max(hit, axis=1, keepdims=True))
        mnns.append(mn if rowval is None else mn * rowval)
    mnn = jnp.concatenate(mnns, axis=0)
    umax5 = jnp.concatenate(umax5s, axis=0)

    best = jnp.max(umax5, axis=1, keepdims=True)
    taken = jnp.zeros((_MUP, 1), jnp.float32)
    rows = []
    for n in range(_N):
        sel = jnp.where(umax5[:, n:n + 1] >= best, 1.0 - taken, 0.0)
        taken = taken + sel
        rows.append(sel * mnn)
    um = jnp.concatenate(rows, axis=1)

    vall_ref[0] = jnp.concatenate(
        [rows[n] * uvt for n in range(_N)] + [um], axis=1)
    vunl_ref[0] = mnn.T
    cnt_ref[0] = jnp.sum(um, axis=0, keepdims=True)


def _main_body(sup_ref, unl_ref, svt_ref, vall_ref, qx_ref, wq_ref, wv_ref,
               wk_ref, vunl_ref, cnt_ref, out_ref):
    ib = pl.program_id(0)
    sup = sup_ref[0]
    unl = unl_ref[0]
    svt = svt_ref[0]
    vall = vall_ref[0]
    qx = qx_ref[0, 0]
    wq = wq_ref[...]
    wv = wv_ref[...]
    wk = wk_ref[...]

    l_sel = jnp.max(cnt_ref[...])
    cnt_b = cnt_ref[pl.ds(ib, 1)].reshape(1, _N)

    qxt = jnp.concatenate(
        [jnp.transpose(qx, (0, 2, 1)),
         jnp.zeros((_QT, _HW2 - _HW, _C), jnp.float32)], axis=1
    ).reshape(_QR, _C)
    qk = _dot(qxt, wq, ((1,), (1,)))
    qv = _dot(qxt, wv, ((1,), (1,)))
    qvn = qv / jnp.maximum(
        jnp.sqrt(jnp.sum(qv * qv, axis=1, keepdims=True)), 1e-12)
    qkk = _dot(qk, wk, ((1,), (0,)))
    qka = jnp.concatenate(
        [qkk, jnp.ones((_QR, 1), jnp.float32)], axis=1)

    biascol = jnp.where(
        jax.lax.broadcasted_iota(jnp.int32, (_MSP, 1), 0) < _MS,
        0.0, -1e30)
    ls = [_dot(qka, jnp.concatenate([sup[n], biascol], axis=1),
               ((1,), (1,))) * _INVSQ for n in range(_N)]
    lu = _dot(qkk, unl, ((1,), (1,))) * _INVSQ

    vu = vunl_ref[0] > 0.0
    rs = jnp.max(ls[0], axis=1, keepdims=True)
    for n in range(1, _N):
        rs = jnp.maximum(rs, jnp.max(ls[n], axis=1, keepdims=True))
    ru = jnp.max(jnp.where(vu, lu, _NEG), axis=1, keepdims=True)
    rmax = jnp.maximum(rs, ru)
    rmax3 = rmax.reshape(_QT, _HW2, 1)
    lu3 = lu.reshape(_QT, _HW2, _MUP)
    cu = jnp.max(lu3, axis=1, keepdims=True)
    mut = jnp.max(jnp.where((lu3 >= rmax3) & (lu3 >= cu) & vu[None],
                            1.0, 0.0), axis=2)
    for n in range(_N):
        ls3 = ls[n].reshape(_QT, _HW2, _MSP)
        cs = jnp.max(ls3, axis=1, keepdims=True)
        mut = jnp.maximum(mut, jnp.max(
            jnp.where((ls3 >= rmax3) & (ls3 >= cs), 1.0, 0.0), axis=2))
    iota2 = jax.lax.broadcasted_iota(jnp.int32, (_QT, _HW2), 1)
    first = jnp.where(iota2 == 0, 1.0, 0.0)
    rowvalid = jnp.where(iota2 < _HW, 1.0, 0.0)
    qm = jnp.where(rmax3[:, :, 0] >= 0.0, mut, first) * rowvalid
    qmf = qm.reshape(_QR, 1)

    lum = lu * qmf
    g = jnp.max(lum, axis=1, keepdims=True)
    lsm = []
    for n in range(_N):
        lsmn = ls[n] * qmf
        lsm.append(lsmn)
        g = jnp.maximum(g, jnp.max(lsmn, axis=1, keepdims=True))
    mp = jnp.maximum(g, 0.0)
    emn = jnp.exp(-mp)
    padfix = (_MSP - _MS) * (1.0 - qmf) * emn
    punl = jnp.exp(lum - mp)
    unl_out = _dot(punl, vall, ((1,), (0,)))

    aligned = []
    for n in range(_N):
        psup = jnp.exp(lsm[n] - mp)
        val = (_dot(psup, svt[n], ((1,), (0,)))
               + unl_out[:, n * _C:(n + 1) * _C])
        padc = l_sel - cnt_b[0, n]
        den = (jnp.sum(psup, axis=1, keepdims=True)
               + unl_out[:, _N * _C + n:_N * _C + n + 1]
               + padc * emn - padfix)
        al = val / den
        al = al / jnp.maximum(
            jnp.sqrt(jnp.sum(al * al, axis=1, keepdims=True)), 1e-12)
        aligned.append(al.reshape(_QT, 1, _HW2, _C))
    alg = jnp.concatenate(aligned, axis=1).reshape(_QT, _N * _HW2, _C)

    s2 = jax.lax.dot_general(alg, qvn.reshape(_QT, _HW2, _C),
                             (((2,), (2,)), ((0,), (0,))),
                             preferred_element_type=jnp.float32)
    s2v = s2.reshape(_QT, _N, _HW2, _HW2)
    irow = jax.lax.broadcasted_iota(jnp.int32, (_QT, _N, _HW2, _HW2), 2)
    topv = jnp.max(jnp.where(irow < _HW, s2v, _NEG), axis=2)
    jcol = jax.lax.broadcasted_iota(jnp.int32, (_QT, _N, _HW2), 2)
    out_ref[0, 0] = jnp.sum(
        jnp.where(jcol < _HW, (topv + 1.0) * 0.5, 0.0), axis=2)


def _loss_body(sim_ref, y_ref, out_ref):
    s = sim_ref[...] * (1.0 / _TEMP)
    m = jnp.max(s, axis=1, keepdims=True)
    lse = m + jnp.log(jnp.sum(jnp.exp(s - m), axis=1, keepdims=True))
    logp = s - lse
    iota = jax.lax.broadcasted_iota(jnp.int32, s.shape, 1)
    picked = jnp.sum(jnp.where(iota == y_ref[...], logp, 0.0),
                     axis=1, keepdims=True)
    out_ref[...] = jnp.sum(-picked / picked.shape[0],
                           axis=0, keepdims=True)


def kernel(support_xf, support_y, query_xf, query_y, unlabeled_xf,
           Wk, Wq, Wv):
    b = support_xf.shape[0]
    q = query_xf.shape[1]
    f32 = jnp.float32

    sup_mc = (support_xf.reshape(b, _N, _K, _C, _HW)
              .transpose(0, 1, 3, 2, 4)
              .reshape(b, _N, _C, _MS)
              .transpose(0, 1, 3, 2))
    sup_mc = jnp.concatenate(
        [sup_mc, jnp.zeros((b, _N, _MSP - _MS, _C), f32)], axis=2)
    unl_mc = (unlabeled_xf.reshape(b, -1, _C, _HW)
              .transpose(0, 2, 1, 3)
              .reshape(b, _C, _MU)
              .transpose(0, 2, 1))
    unl_mc = jnp.concatenate(
        [unl_mc, jnp.zeros((b, _MUP - _MU, _C), f32)], axis=1)
    qx5 = query_xf.reshape(b, _NQT, _QT, _C, _HW)

    svt, vall, vunl, cnt = pl.pallas_call(
        _select_body,
        grid=(b,),
        in_specs=[
            pl.BlockSpec((1, _N, _MSP, _C), lambda i: (i, 0, 0, 0)),
            pl.BlockSpec((1, _MUP, _C), lambda i: (i, 0, 0)),
            pl.BlockSpec((_C, _C), lambda i: (0, 0)),
        ],
        out_specs=[
            pl.BlockSpec((1, _N, _MSP, _C), lambda i: (i, 0, 0, 0)),
            pl.BlockSpec((1, _MUP, _VW), lambda i: (i, 0, 0)),
            pl.BlockSpec((1, 1, _MUP), lambda i: (i, 0, 0)),
            pl.BlockSpec((1, 1, _N), lambda i: (i, 0, 0)),
        ],
        out_shape=[
            jax.ShapeDtypeStruct((b, _N, _MSP, _C), f32),
            jax.ShapeDtypeStruct((b, _MUP, _VW), f32),
            jax.ShapeDtypeStruct((b, 1, _MUP), f32),
            jax.ShapeDtypeStruct((b, 1, _N), f32),
        ],
        compiler_params=pltpu.CompilerParams(
            dimension_semantics=("arbitrary",)),
    )(sup_mc, unl_mc, Wv)

    sim = pl.pallas_call(
        _main_body,
        grid=(b, _NQT),
        in_specs=[
            pl.BlockSpec((1, _N, _MSP, _C), lambda i, j: (i, 0, 0, 0)),
            pl.BlockSpec((1, _MUP, _C), lambda i, j: (i, 0, 0)),
            pl.BlockSpec((1, _N, _MSP, _C), lambda i, j: (i, 0, 0, 0)),
            pl.BlockSpec((1, _MUP, _VW), lambda i, j: (i, 0, 0)),
            pl.BlockSpec((1, 1, _QT, _C, _HW), lambda i, j: (i, j, 0, 0, 0)),
            pl.BlockSpec((_C, _C), lambda i, j: (0, 0)),
            pl.BlockSpec((_C, _C), lambda i, j: (0, 0)),
            pl.BlockSpec((_C, _C), lambda i, j: (0, 0)),
            pl.BlockSpec((1, 1, _MUP), lambda i, j: (i, 0, 0)),
            pl.BlockSpec((b, 1, _N), lambda i, j: (0, 0, 0)),
        ],
        out_specs=pl.BlockSpec((1, 1, _QT, _N), lambda i, j: (i, j, 0, 0)),
        out_shape=jax.ShapeDtypeStruct((b, _NQT, _QT, _N), f32),
        compiler_params=pltpu.CompilerParams(
            dimension_semantics=("arbitrary", "arbitrary")),
    )(sup_mc, unl_mc, svt, vall, qx5, Wq, Wv, Wk, vunl, cnt)

    loss = pl.pallas_call(
        _loss_body,
        grid=(1,),
        in_specs=[
            pl.BlockSpec((b * q, _N), lambda i: (0, 0)),
            pl.BlockSpec((b * q, 1), lambda i: (0, 0)),
        ],
        out_specs=pl.BlockSpec((1, 1), lambda i: (0, 0)),
        out_shape=jax.ShapeDtypeStruct((1, 1), f32),
    )(sim.reshape(b * q, _N), query_y.reshape(b * q, 1).astype(jnp.int32))

    return loss.reshape(())


# speedup vs baseline: 1.5839x; 1.5839x over previous
import jax
import jax.numpy as jnp
from jax.experimental import pallas as pl
from jax.experimental.pallas import tpu as pltpu

_N = 5
_K = 5
_C = 64
_HW = 25
_MS = _K * _HW
_MST = _N * _MS
_MU = 2500
_QT = 15
_NQT = 5
_INVSQ = 0.125
_TEMP = 2.0
_NEG = -jnp.inf
_VW = _N * _C + _N


def _dot(a, b, dims):
    return jax.lax.dot_general(a, b, (dims, ((), ())),
                               preferred_element_type=jnp.float32)


def _select_body(sup_ref, unl_ref, wk_ref, wv_ref,
                 skt_ref, svt_ref, ukt_ref, vall_ref, vunl_ref, cnt_ref):
    sup = sup_ref[0]
    unl = unl_ref[0]
    wk = wk_ref[...]
    wv = wv_ref[...]
    cn = ((1,), (1,))
    skt_ref[0] = _dot(sup, wk, cn).reshape(_N, _MS, _C)
    svt_ref[0] = _dot(sup, wv, cn).reshape(_N, _MS, _C)
    ukt_ref[0] = _dot(unl, wk, cn)
    uvt = _dot(unl, wv, cn)

    sn = sup / jnp.maximum(
        jnp.sqrt(jnp.sum(sup * sup, axis=1, keepdims=True)), 1e-12)
    sn3 = sn.reshape(_N, _MS, _C)
    un = unl / jnp.maximum(
        jnp.sqrt(jnp.sum(unl * unl, axis=1, keepdims=True)), 1e-12)

    ch = 625
    nch = _MU // ch
    rowmaxs, unears, umax5s = [], [], []
    colmax = [jnp.full((1, _MS), _NEG, jnp.float32) for _ in range(_N)]
    iota = jax.lax.broadcasted_iota(jnp.int32, (ch, _MS), 1)
    for ci in range(nch):
        uc = un[ci * ch:(ci + 1) * ch]
        ccs = [_dot(uc, sn3[n], cn) for n in range(_N)]
        umax5 = jnp.concatenate(
            [jnp.max(ccs[n], axis=1, keepdims=True) for n in range(_N)],
            axis=1)
        rm = jnp.max(umax5, axis=1, keepdims=True)
        unear = jnp.full((ch, 1), _MST, jnp.int32)
        for n in range(_N):
            cand = jnp.min(jnp.where(ccs[n] >= rm, iota + n * _MS, _MST),
                           axis=1, keepdims=True)
            unear = jnp.minimum(unear, cand)
            colmax[n] = jnp.maximum(colmax[n],
                                    jnp.max(ccs[n], axis=0, keepdims=True))
        rowmaxs.append(rm)
        unears.append(unear)
        umax5s.append(umax5)

    mnns = []
    for ci in range(nch):
        cg = jnp.full((ch, 1), _NEG, jnp.float32)
        for n in range(_N):
            hit = jnp.where(unears[ci] == iota + n * _MS, colmax[n], _NEG)
            cg = jnp.maximum(cg, jnp.max(hit, axis=1, keepdims=True))
        mnns.append((rowmaxs[ci] >= cg).astype(jnp.float32))
    mnn = jnp.concatenate(mnns, axis=0)
    umax5 = jnp.concatenate(umax5s, axis=0)

    best = jnp.max(umax5, axis=1, keepdims=True)
    taken = jnp.zeros((_MU, 1), jnp.float32)
    rows = []
    for n in range(_N):
        sel = jnp.where(umax5[:, n:n + 1] >= best, 1.0 - taken, 0.0)
        taken = taken + sel
        rows.append(sel * mnn)
    um = jnp.concatenate(rows, axis=1)

    vall_ref[0] = jnp.concatenate(
        [rows[n] * uvt for n in range(_N)] + [um], axis=1)
    vunl_ref[0] = mnn.T
    cnt_ref[0] = jnp.sum(um, axis=0, keepdims=True)


def _main_body(skt_ref, svt_ref, ukt_ref, vall_ref, qx_ref, wq_ref, wv_ref,
               vunl_ref, cnt_ref, out_ref):
    ib = pl.program_id(0)
    skt = skt_ref[0]
    svt = svt_ref[0]
    ukt = ukt_ref[0]
    vall = vall_ref[0]
    qx = qx_ref[0, 0]
    wq = wq_ref[...]
    wv = wv_ref[...]

    l_sel = jnp.max(cnt_ref[...])
    cnt_b = cnt_ref[pl.ds(ib, 1)].reshape(1, _N)

    hw2 = 32
    qr = _QT * hw2
    qxt = jnp.concatenate(
        [jnp.transpose(qx, (0, 2, 1)),
         jnp.zeros((_QT, hw2 - _HW, _C), jnp.float32)], axis=1
    ).reshape(qr, _C)
    qk = _dot(qxt, wq, ((1,), (1,)))
    qv = _dot(qxt, wv, ((1,), (1,)))
    qvn = qv / jnp.maximum(
        jnp.sqrt(jnp.sum(qv * qv, axis=1, keepdims=True)), 1e-12)

    ls = [_dot(qk, skt[n], ((1,), (1,))) * _INVSQ for n in range(_N)]
    lu = _dot(qk, ukt, ((1,), (1,))) * _INVSQ

    vu = vunl_ref[0] > 0.0
    rs = jnp.max(ls[0], axis=1, keepdims=True)
    for n in range(1, _N):
        rs = jnp.maximum(rs, jnp.max(ls[n], axis=1, keepdims=True))
    ru = jnp.max(jnp.where(vu, lu, _NEG), axis=1, keepdims=True)
    rmax = jnp.maximum(rs, ru)
    rmax3 = rmax.reshape(_QT, hw2, 1)
    lu3 = lu.reshape(_QT, hw2, _MU)
    cu = jnp.max(lu3, axis=1, keepdims=True)
    mut = jnp.max(jnp.where((lu3 >= rmax3) & (lu3 >= cu) & vu[None],
                            1.0, 0.0), axis=2)
    for n in range(_N):
        ls3 = ls[n].reshape(_QT, hw2, _MS)
        cs = jnp.max(ls3, axis=1, keepdims=True)
        mut = jnp.maximum(mut, jnp.max(
            jnp.where((ls3 >= rmax3) & (ls3 >= cs), 1.0, 0.0), axis=2))
    iota2 = jax.lax.broadcasted_iota(jnp.int32, (_QT, hw2), 1)
    first = jnp.where(iota2 == 0, 1.0, 0.0)
    rowvalid = jnp.where(iota2 < _HW, 1.0, 0.0)
    qm = jnp.where(rmax3[:, :, 0] >= 0.0, mut, first) * rowvalid
    qmf = qm.reshape(qr, 1)

    lum = lu * qmf
    g = jnp.max(lum, axis=1, keepdims=True)
    lsm = []
    for n in range(_N):
        lsmn = ls[n] * qmf
        lsm.append(lsmn)
        g = jnp.maximum(g, jnp.max(lsmn, axis=1, keepdims=True))
    mp = jnp.maximum(g, 0.0)
    emn = jnp.exp(-mp)
    punl = jnp.exp(lum - mp)
    unl_out = _dot(punl, vall, ((1,), (0,)))

    aligned = []
    for n in range(_N):
        psup = jnp.exp(lsm[n] - mp)
        val = (_dot(psup, svt[n], ((1,), (0,)))
               + unl_out[:, n * _C:(n + 1) * _C])
        padc = l_sel - cnt_b[0, n]
        den = (jnp.sum(psup, axis=1, keepdims=True)
               + unl_out[:, _N * _C + n:_N * _C + n + 1]
               + padc * emn)
        al = val / den
        al = al / jnp.maximum(
            jnp.sqrt(jnp.sum(al * al, axis=1, keepdims=True)), 1e-12)
        aligned.append(al.reshape(_QT, 1, hw2, _C))
    alg = jnp.concatenate(aligned, axis=1).reshape(_QT, _N * hw2, _C)

    s2 = jax.lax.dot_general(alg, qvn.reshape(_QT, hw2, _C),
                             (((2,), (2,)), ((0,), (0,))),
                             preferred_element_type=jnp.float32)
    s2v = s2.reshape(_QT, _N, hw2, hw2)
    irow = jax.lax.broadcasted_iota(jnp.int32, (_QT, _N, hw2, hw2), 2)
    topv = jnp.max(jnp.where(irow < _HW, s2v, _NEG), axis=2)
    jcol = jax.lax.broadcasted_iota(jnp.int32, (_QT, _N, hw2), 2)
    out_ref[0, 0] = jnp.sum(
        jnp.where(jcol < _HW, (topv + 1.0) * 0.5, 0.0), axis=2)


def _loss_body(sim_ref, y_ref, out_ref):
    s = sim_ref[...] * (1.0 / _TEMP)
    m = jnp.max(s, axis=1, keepdims=True)
    lse = m + jnp.log(jnp.sum(jnp.exp(s - m), axis=1, keepdims=True))
    logp = s - lse
    iota = jax.lax.broadcasted_iota(jnp.int32, s.shape, 1)
    picked = jnp.sum(jnp.where(iota == y_ref[...], logp, 0.0),
                     axis=1, keepdims=True)
    out_ref[...] = jnp.sum(-picked / picked.shape[0],
                           axis=0, keepdims=True)


def kernel(support_xf, support_y, query_xf, query_y, unlabeled_xf,
           Wk, Wq, Wv):
    b = support_xf.shape[0]
    q = query_xf.shape[1]
    f32 = jnp.float32

    sup_mc = (support_xf.reshape(b, _N, _K, _C, _HW)
              .transpose(0, 1, 3, 2, 4)
              .reshape(b, _N, _C, _MS)
              .transpose(0, 1, 3, 2)
              .reshape(b, _MST, _C))
    unl_mc = (unlabeled_xf.reshape(b, -1, _C, _HW)
              .transpose(0, 2, 1, 3)
              .reshape(b, _C, _MU)
              .transpose(0, 2, 1))
    qx5 = query_xf.reshape(b, _NQT, _QT, _C, _HW)

    skt, svt, ukt, vall, vunl, cnt = pl.pallas_call(
        _select_body,
        grid=(b,),
        in_specs=[
            pl.BlockSpec((1, _MST, _C), lambda i: (i, 0, 0)),
            pl.BlockSpec((1, _MU, _C), lambda i: (i, 0, 0)),
            pl.BlockSpec((_C, _C), lambda i: (0, 0)),
            pl.BlockSpec((_C, _C), lambda i: (0, 0)),
        ],
        out_specs=[
            pl.BlockSpec((1, _N, _MS, _C), lambda i: (i, 0, 0, 0)),
            pl.BlockSpec((1, _N, _MS, _C), lambda i: (i, 0, 0, 0)),
            pl.BlockSpec((1, _MU, _C), lambda i: (i, 0, 0)),
            pl.BlockSpec((1, _MU, _VW), lambda i: (i, 0, 0)),
            pl.BlockSpec((1, 1, _MU), lambda i: (i, 0, 0)),
            pl.BlockSpec((1, 1, _N), lambda i: (i, 0, 0)),
        ],
        out_shape=[
            jax.ShapeDtypeStruct((b, _N, _MS, _C), f32),
            jax.ShapeDtypeStruct((b, _N, _MS, _C), f32),
            jax.ShapeDtypeStruct((b, _MU, _C), f32),
            jax.ShapeDtypeStruct((b, _MU, _VW), f32),
            jax.ShapeDtypeStruct((b, 1, _MU), f32),
            jax.ShapeDtypeStruct((b, 1, _N), f32),
        ],
        compiler_params=pltpu.CompilerParams(
            dimension_semantics=("arbitrary",)),
    )(sup_mc, unl_mc, Wk, Wv)

    sim = pl.pallas_call(
        _main_body,
        grid=(b, _NQT),
        in_specs=[
            pl.BlockSpec((1, _N, _MS, _C), lambda i, j: (i, 0, 0, 0)),
            pl.BlockSpec((1, _N, _MS, _C), lambda i, j: (i, 0, 0, 0)),
            pl.BlockSpec((1, _MU, _C), lambda i, j: (i, 0, 0)),
            pl.BlockSpec((1, _MU, _VW), lambda i, j: (i, 0, 0)),
            pl.BlockSpec((1, 1, _QT, _C, _HW), lambda i, j: (i, j, 0, 0, 0)),
            pl.BlockSpec((_C, _C), lambda i, j: (0, 0)),
            pl.BlockSpec((_C, _C), lambda i, j: (0, 0)),
            pl.BlockSpec((1, 1, _MU), lambda i, j: (i, 0, 0)),
            pl.BlockSpec((b, 1, _N), lambda i, j: (0, 0, 0)),
        ],
        out_specs=pl.BlockSpec((1, 1, _QT, _N), lambda i, j: (i, j, 0, 0)),
        out_shape=jax.ShapeDtypeStruct((b, _NQT, _QT, _N), f32),
        compiler_params=pltpu.CompilerParams(
            dimension_semantics=("arbitrary", "arbitrary")),
    )(skt, svt, ukt, vall, qx5, Wq, Wv, vunl, cnt)

    loss = pl.pallas_call(
        _loss_body,
        grid=(1,),
        in_specs=[
            pl.BlockSpec((b * q, _N), lambda i: (0, 0)),
            pl.BlockSpec((b * q, 1), lambda i: (0, 0)),
        ],
        out_specs=pl.BlockSpec((1, 1), lambda i: (0, 0)),
        out_shape=jax.ShapeDtypeStruct((1, 1), f32),
    )(sim.reshape(b * q, _N), query_y.reshape(b * q, 1).astype(jnp.int32))

    return loss.reshape(())


# speedup vs baseline: 1.5881x; 1.0027x over previous
import jax
import jax.numpy as jnp
from jax.experimental import pallas as pl
from jax.experimental.pallas import tpu as pltpu

_N = 5
_K = 5
_C = 64
_HW = 25
_MS = _K * _HW
_MST = _N * _MS
_MU = 2500
_QT = 15
_NQT = 5
_INVSQ = 0.125
_TEMP = 2.0
_NEG = -jnp.inf
_VW = _N * _C + _N


def _dot(a, b, dims):
    return jax.lax.dot_general(a, b, (dims, ((), ())),
                               preferred_element_type=jnp.float32)


def _select_body(sup_ref, unl_ref, wk_ref, wv_ref,
                 skt_ref, svt_ref, ukt_ref, vall_ref, vunl_ref, cnt_ref):
    sup = sup_ref[0]
    unl = unl_ref[0]
    wk = wk_ref[...]
    wv = wv_ref[...]
    cn = ((1,), (1,))
    skt_ref[0] = _dot(sup, wk, cn).reshape(_N, _MS, _C)
    svt_ref[0] = _dot(sup, wv, cn).reshape(_N, _MS, _C)
    ukt_ref[0] = _dot(unl, wk, cn)
    uvt = _dot(unl, wv, cn)

    sn = sup / jnp.maximum(
        jnp.sqrt(jnp.sum(sup * sup, axis=1, keepdims=True)), 1e-12)
    sn3 = sn.reshape(_N, _MS, _C)
    un = unl / jnp.maximum(
        jnp.sqrt(jnp.sum(unl * unl, axis=1, keepdims=True)), 1e-12)

    ch = 625
    nch = _MU // ch
    rowmaxs, unears, umax5s = [], [], []
    colmax = [jnp.full((1, _MS), _NEG, jnp.float32) for _ in range(_N)]
    iota = jax.lax.broadcasted_iota(jnp.int32, (ch, _MS), 1)
    for ci in range(nch):
        uc = un[ci * ch:(ci + 1) * ch]
        ccs = [_dot(uc, sn3[n], cn) for n in range(_N)]
        umax5 = jnp.concatenate(
            [jnp.max(ccs[n], axis=1, keepdims=True) for n in range(_N)],
            axis=1)
        rm = jnp.max(umax5, axis=1, keepdims=True)
        unear = jnp.full((ch, 1), _MST, jnp.int32)
        for n in range(_N):
            cand = jnp.min(jnp.where(ccs[n] >= rm, iota + n * _MS, _MST),
                           axis=1, keepdims=True)
            unear = jnp.minimum(unear, cand)
            colmax[n] = jnp.maximum(colmax[n],
                                    jnp.max(ccs[n], axis=0, keepdims=True))
        rowmaxs.append(rm)
        unears.append(unear)
        umax5s.append(umax5)

    mnns = []
    for ci in range(nch):
        cg = jnp.full((ch, 1), _NEG, jnp.float32)
        for n in range(_N):
            hit = jnp.where(unears[ci] == iota + n * _MS, colmax[n], _NEG)
            cg = jnp.maximum(cg, jnp.max(hit, axis=1, keepdims=True))
        mnns.append((rowmaxs[ci] >= cg).astype(jnp.float32))
    mnn = jnp.concatenate(mnns, axis=0)
    umax5 = jnp.concatenate(umax5s, axis=0)

    best = jnp.max(umax5, axis=1, keepdims=True)
    taken = jnp.zeros((_MU, 1), jnp.float32)
    rows = []
    for n in range(_N):
        sel = jnp.where(umax5[:, n:n + 1] >= best, 1.0 - taken, 0.0)
        taken = taken + sel
        rows.append(sel * mnn)
    um = jnp.concatenate(rows, axis=1)

    vall_ref[0] = jnp.concatenate(
        [rows[n] * uvt for n in range(_N)] + [um], axis=1)
    vunl_ref[0] = mnn.T
    cnt_ref[0] = jnp.sum(um, axis=0, keepdims=True)


def _main_body(skt_ref, svt_ref, ukt_ref, vall_ref, qx_ref, wq_ref, wv_ref,
               vunl_ref, cnt_ref, out_ref):
    ib = pl.program_id(0)
    skt = skt_ref[0]
    svt = svt_ref[0]
    ukt = ukt_ref[0]
    vall = vall_ref[0]
    qx = qx_ref[0, 0]
    wq = wq_ref[...]
    wv = wv_ref[...]

    l_sel = jnp.max(cnt_ref[...])
    cnt_b = cnt_ref[pl.ds(ib, 1)].reshape(1, _N)

    hw2 = 32
    qr = _QT * hw2
    qxt = jnp.concatenate(
        [jnp.transpose(qx, (0, 2, 1)),
         jnp.zeros((_QT, hw2 - _HW, _C), jnp.float32)], axis=1
    ).reshape(qr, _C)
    qk = _dot(qxt, wq, ((1,), (1,)))
    qv = _dot(qxt, wv, ((1,), (1,)))
    qvn = qv / jnp.maximum(
        jnp.sqrt(jnp.sum(qv * qv, axis=1, keepdims=True)), 1e-12)

    ls = [_dot(qk, skt[n], ((1,), (1,))) * _INVSQ for n in range(_N)]
    lu = _dot(qk, ukt, ((1,), (1,))) * _INVSQ

    vu = vunl_ref[0] > 0.0
    rs = jnp.max(ls[0], axis=1, keepdims=True)
    for n in range(1, _N):
        rs = jnp.maximum(rs, jnp.max(ls[n], axis=1, keepdims=True))
    ru = jnp.max(jnp.where(vu, lu, _NEG), axis=1, keepdims=True)
    rmax = jnp.maximum(rs, ru)
    rmax3 = rmax.reshape(_QT, hw2, 1)
    lu3 = lu.reshape(_QT, hw2, _MU)
    cu = jnp.max(lu3, axis=1, keepdims=True)
    mut = jnp.max(jnp.where((lu3 >= rmax3) & (lu3 >= cu) & vu[None],
                            1.0, 0.0), axis=2)
    for n in range(_N):
        ls3 = ls[n].reshape(_QT, hw2, _MS)
        cs = jnp.max(ls3, axis=1, keepdims=True)
        mut = jnp.maximum(mut, jnp.max(
            jnp.where((ls3 >= rmax3) & (ls3 >= cs), 1.0, 0.0), axis=2))
    iota2 = jax.lax.broadcasted_iota(jnp.int32, (_QT, hw2), 1)
    first = jnp.where(iota2 == 0, 1.0, 0.0)
    rowvalid = jnp.where(iota2 < _HW, 1.0, 0.0)
    qm = jnp.where(rmax3[:, :, 0] >= 0.0, mut, first) * rowvalid
    qmf = qm.reshape(qr, 1)

    lum = lu * qmf
    g = jnp.max(lum, axis=1, keepdims=True)
    lsm = []
    for n in range(_N):
        lsmn = ls[n] * qmf
        lsm.append(lsmn)
        g = jnp.maximum(g, jnp.max(lsmn, axis=1, keepdims=True))
    mp = jnp.maximum(g, 0.0)
    emn = jnp.exp(-mp)
    punl = jnp.exp(lum - mp)
    unl_out = _dot(punl, vall, ((1,), (0,)))

    aligned = []
    for n in range(_N):
        psup = jnp.exp(lsm[n] - mp)
        val = (_dot(psup, svt[n], ((1,), (0,)))
               + unl_out[:, n * _C:(n + 1) * _C])
        padc = l_sel - cnt_b[0, n]
        den = (jnp.sum(psup, axis=1, keepdims=True)
               + unl_out[:, _N * _C + n:_N * _C + n + 1]
               + padc * emn)
        al = val / den
        al = al / jnp.maximum(
            jnp.sqrt(jnp.sum(al * al, axis=1, keepdims=True)), 1e-12)
        aligned.append(al.reshape(_QT, 1, hw2, _C))
    alg = jnp.concatenate(aligned, axis=1).reshape(_QT, _N * hw2, _C)

    s2 = jax.lax.dot_general(alg, qvn.reshape(_QT, hw2, _C),
                             (((2,), (2,)), ((0,), (0,))),
                             preferred_element_type=jnp.float32)
    s2v = s2.reshape(_QT, _N, hw2, hw2)
    irow = jax.lax.broadcasted_iota(jnp.int32, (_QT, _N, hw2, hw2), 2)
    topv = jnp.max(jnp.where(irow < _HW, s2v, _NEG), axis=2)
    jcol = jax.lax.broadcasted_iota(jnp.int32, (_QT, _N, hw2), 2)
    out_ref[0, 0] = jnp.sum(
        jnp.where(jcol < _HW, (topv + 1.0) * 0.5, 0.0), axis=2)


def _loss_body(sim_ref, y_ref, out_ref):
    s = sim_ref[...] * (1.0 / _TEMP)
    m = jnp.max(s, axis=1, keepdims=True)
    lse = m + jnp.log(jnp.sum(jnp.exp(s - m), axis=1, keepdims=True))
    logp = s - lse
    iota = jax.lax.broadcasted_iota(jnp.int32, s.shape, 1)
    picked = jnp.sum(jnp.where(iota == y_ref[...], logp, 0.0),
                     axis=1, keepdims=True)
    out_ref[...] = jnp.sum(-picked / picked.shape[0],
                           axis=0, keepdims=True)


def kernel(support_xf, support_y, query_xf, query_y, unlabeled_xf,
           Wk, Wq, Wv):
    b = support_xf.shape[0]
    q = query_xf.shape[1]
    f32 = jnp.float32

    sup_mc = (support_xf.reshape(b, _N, _K, _C, _HW)
              .transpose(0, 1, 3, 2, 4)
              .reshape(b, _N, _C, _MS)
              .transpose(0, 1, 3, 2)
              .reshape(b, _MST, _C))
    unl_mc = (unlabeled_xf.reshape(b, -1, _C, _HW)
              .transpose(0, 2, 1, 3)
              .reshape(b, _C, _MU)
              .transpose(0, 2, 1))
    qx5 = query_xf.reshape(b, _NQT, _QT, _C, _HW)

    skt, svt, ukt, vall, vunl, cnt = pl.pallas_call(
        _select_body,
        grid=(b,),
        in_specs=[
            pl.BlockSpec((1, _MST, _C), lambda i: (i, 0, 0)),
            pl.BlockSpec((1, _MU, _C), lambda i: (i, 0, 0)),
            pl.BlockSpec((_C, _C), lambda i: (0, 0)),
            pl.BlockSpec((_C, _C), lambda i: (0, 0)),
        ],
        out_specs=[
            pl.BlockSpec((1, _N, _MS, _C), lambda i: (i, 0, 0, 0)),
            pl.BlockSpec((1, _N, _MS, _C), lambda i: (i, 0, 0, 0)),
            pl.BlockSpec((1, _MU, _C), lambda i: (i, 0, 0)),
            pl.BlockSpec((1, _MU, _VW), lambda i: (i, 0, 0)),
            pl.BlockSpec((1, 1, _MU), lambda i: (i, 0, 0)),
            pl.BlockSpec((1, 1, _N), lambda i: (i, 0, 0)),
        ],
        out_shape=[
            jax.ShapeDtypeStruct((b, _N, _MS, _C), f32),
            jax.ShapeDtypeStruct((b, _N, _MS, _C), f32),
            jax.ShapeDtypeStruct((b, _MU, _C), f32),
            jax.ShapeDtypeStruct((b, _MU, _VW), f32),
            jax.ShapeDtypeStruct((b, 1, _MU), f32),
            jax.ShapeDtypeStruct((b, 1, _N), f32),
        ],
        compiler_params=pltpu.CompilerParams(
            dimension_semantics=("parallel",)),
    )(sup_mc, unl_mc, Wk, Wv)

    sim = pl.pallas_call(
        _main_body,
        grid=(b, _NQT),
        in_specs=[
            pl.BlockSpec((1, _N, _MS, _C), lambda i, j: (i, 0, 0, 0)),
            pl.BlockSpec((1, _N, _MS, _C), lambda i, j: (i, 0, 0, 0)),
            pl.BlockSpec((1, _MU, _C), lambda i, j: (i, 0, 0)),
            pl.BlockSpec((1, _MU, _VW), lambda i, j: (i, 0, 0)),
            pl.BlockSpec((1, 1, _QT, _C, _HW), lambda i, j: (i, j, 0, 0, 0)),
            pl.BlockSpec((_C, _C), lambda i, j: (0, 0)),
            pl.BlockSpec((_C, _C), lambda i, j: (0, 0)),
            pl.BlockSpec((1, 1, _MU), lambda i, j: (i, 0, 0)),
            pl.BlockSpec((b, 1, _N), lambda i, j: (0, 0, 0)),
        ],
        out_specs=pl.BlockSpec((1, 1, _QT, _N), lambda i, j: (i, j, 0, 0)),
        out_shape=jax.ShapeDtypeStruct((b, _NQT, _QT, _N), f32),
        compiler_params=pltpu.CompilerParams(
            dimension_semantics=("parallel", "arbitrary")),
    )(skt, svt, ukt, vall, qx5, Wq, Wv, vunl, cnt)

    loss = pl.pallas_call(
        _loss_body,
        grid=(1,),
        in_specs=[
            pl.BlockSpec((b * q, _N), lambda i: (0, 0)),
            pl.BlockSpec((b * q, 1), lambda i: (0, 0)),
        ],
        out_specs=pl.BlockSpec((1, 1), lambda i: (0, 0)),
        out_shape=jax.ShapeDtypeStruct((1, 1), f32),
    )(sim.reshape(b * q, _N), query_y.reshape(b * q, 1).astype(jnp.int32))

    return loss.reshape(())


# speedup vs baseline: 1.6418x; 1.0338x over previous
import jax
import jax.numpy as jnp
from jax.experimental import pallas as pl
from jax.experimental.pallas import tpu as pltpu

_N = 5
_K = 5
_C = 64
_HW = 25
_MS = _K * _HW
_MST = _N * _MS
_MU = 2500
_QT = 25
_NQT = 3
_INVSQ = 0.125
_TEMP = 2.0
_NEG = -jnp.inf
_VW = _N * _C + _N


def _dot(a, b, dims):
    return jax.lax.dot_general(a, b, (dims, ((), ())),
                               preferred_element_type=jnp.float32)


def _select_body(sup_ref, unl_ref, wk_ref, wv_ref,
                 skt_ref, svt_ref, ukt_ref, vall_ref, vunl_ref, cnt_ref):
    sup = sup_ref[0]
    unl = unl_ref[0]
    wk = wk_ref[...]
    wv = wv_ref[...]
    cn = ((1,), (1,))
    skt_ref[0] = _dot(sup, wk, cn).reshape(_N, _MS, _C)
    svt_ref[0] = _dot(sup, wv, cn).reshape(_N, _MS, _C)
    ukt_ref[0] = _dot(unl, wk, cn)
    uvt = _dot(unl, wv, cn)

    sn = sup / jnp.maximum(
        jnp.sqrt(jnp.sum(sup * sup, axis=1, keepdims=True)), 1e-12)
    sn3 = sn.reshape(_N, _MS, _C)
    un = unl / jnp.maximum(
        jnp.sqrt(jnp.sum(unl * unl, axis=1, keepdims=True)), 1e-12)

    ch = 625
    nch = _MU // ch
    rowmaxs, unears, umax5s = [], [], []
    colmax = [jnp.full((1, _MS), _NEG, jnp.float32) for _ in range(_N)]
    iota = jax.lax.broadcasted_iota(jnp.int32, (ch, _MS), 1)
    for ci in range(nch):
        uc = un[ci * ch:(ci + 1) * ch]
        ccs = [_dot(uc, sn3[n], cn) for n in range(_N)]
        umax5 = jnp.concatenate(
            [jnp.max(ccs[n], axis=1, keepdims=True) for n in range(_N)],
            axis=1)
        rm = jnp.max(umax5, axis=1, keepdims=True)
        unear = jnp.full((ch, 1), _MST, jnp.int32)
        for n in range(_N):
            cand = jnp.min(jnp.where(ccs[n] >= rm, iota + n * _MS, _MST),
                           axis=1, keepdims=True)
            unear = jnp.minimum(unear, cand)
            colmax[n] = jnp.maximum(colmax[n],
                                    jnp.max(ccs[n], axis=0, keepdims=True))
        rowmaxs.append(rm)
        unears.append(unear)
        umax5s.append(umax5)

    mnns = []
    for ci in range(nch):
        cg = jnp.full((ch, 1), _NEG, jnp.float32)
        for n in range(_N):
            hit = jnp.where(unears[ci] == iota + n * _MS, colmax[n], _NEG)
            cg = jnp.maximum(cg, jnp.max(hit, axis=1, keepdims=True))
        mnns.append((rowmaxs[ci] >= cg).astype(jnp.float32))
    mnn = jnp.concatenate(mnns, axis=0)
    umax5 = jnp.concatenate(umax5s, axis=0)

    best = jnp.max(umax5, axis=1, keepdims=True)
    taken = jnp.zeros((_MU, 1), jnp.float32)
    rows = []
    for n in range(_N):
        sel = jnp.where(umax5[:, n:n + 1] >= best, 1.0 - taken, 0.0)
        taken = taken + sel
        rows.append(sel * mnn)
    um = jnp.concatenate(rows, axis=1)

    vall_ref[0] = jnp.concatenate(
        [rows[n] * uvt for n in range(_N)] + [um], axis=1)
    vunl_ref[0] = mnn.T
    cnt_ref[0] = jnp.sum(um, axis=0, keepdims=True)


def _main_body(skt_ref, svt_ref, ukt_ref, vall_ref, qx_ref, wq_ref, wv_ref,
               vunl_ref, cnt_ref, out_ref):
    ib = pl.program_id(0)
    skt = skt_ref[0]
    svt = svt_ref[0]
    ukt = ukt_ref[0]
    vall = vall_ref[0]
    qx = qx_ref[0, 0]
    wq = wq_ref[...]
    wv = wv_ref[...]

    l_sel = jnp.max(cnt_ref[...])
    cnt_b = cnt_ref[pl.ds(ib, 1)].reshape(1, _N)

    hw2 = 32
    qr = _QT * hw2
    qxt = jnp.concatenate(
        [jnp.transpose(qx, (0, 2, 1)),
         jnp.zeros((_QT, hw2 - _HW, _C), jnp.float32)], axis=1
    ).reshape(qr, _C)
    qk = _dot(qxt, wq, ((1,), (1,)))
    qv = _dot(qxt, wv, ((1,), (1,)))
    qvn = qv / jnp.maximum(
        jnp.sqrt(jnp.sum(qv * qv, axis=1, keepdims=True)), 1e-12)

    ls = [_dot(qk, skt[n], ((1,), (1,))) * _INVSQ for n in range(_N)]
    lu = _dot(qk, ukt, ((1,), (1,))) * _INVSQ

    vu = vunl_ref[0] > 0.0
    rs = jnp.max(ls[0], axis=1, keepdims=True)
    for n in range(1, _N):
        rs = jnp.maximum(rs, jnp.max(ls[n], axis=1, keepdims=True))
    ru = jnp.max(jnp.where(vu, lu, _NEG), axis=1, keepdims=True)
    rmax = jnp.maximum(rs, ru)
    rmax3 = rmax.reshape(_QT, hw2, 1)
    lu3 = lu.reshape(_QT, hw2, _MU)
    cu = jnp.max(lu3, axis=1, keepdims=True)
    mut = jnp.max(jnp.where((lu3 >= rmax3) & (lu3 >= cu) & vu[None],
                            1.0, 0.0), axis=2)
    for n in range(_N):
        ls3 = ls[n].reshape(_QT, hw2, _MS)
        cs = jnp.max(ls3, axis=1, keepdims=True)
        mut = jnp.maximum(mut, jnp.max(
            jnp.where((ls3 >= rmax3) & (ls3 >= cs), 1.0, 0.0), axis=2))
    iota2 = jax.lax.broadcasted_iota(jnp.int32, (_QT, hw2), 1)
    first = jnp.where(iota2 == 0, 1.0, 0.0)
    rowvalid = jnp.where(iota2 < _HW, 1.0, 0.0)
    qm = jnp.where(rmax3[:, :, 0] >= 0.0, mut, first) * rowvalid
    qmf = qm.reshape(qr, 1)

    lum = lu * qmf
    g = jnp.max(lum, axis=1, keepdims=True)
    lsm = []
    for n in range(_N):
        lsmn = ls[n] * qmf
        lsm.append(lsmn)
        g = jnp.maximum(g, jnp.max(lsmn, axis=1, keepdims=True))
    mp = jnp.maximum(g, 0.0)
    emn = jnp.exp(-mp)
    punl = jnp.exp(lum - mp)
    unl_out = _dot(punl, vall, ((1,), (0,)))

    aligned = []
    for n in range(_N):
        psup = jnp.exp(lsm[n] - mp)
        val = (_dot(psup, svt[n], ((1,), (0,)))
               + unl_out[:, n * _C:(n + 1) * _C])
        padc = l_sel - cnt_b[0, n]
        den = (jnp.sum(psup, axis=1, keepdims=True)
               + unl_out[:, _N * _C + n:_N * _C + n + 1]
               + padc * emn)
        al = val / den
        al = al / jnp.maximum(
            jnp.sqrt(jnp.sum(al * al, axis=1, keepdims=True)), 1e-12)
        aligned.append(al.reshape(_QT, 1, hw2, _C))
    alg = jnp.concatenate(aligned, axis=1).reshape(_QT, _N * hw2, _C)

    s2 = jax.lax.dot_general(alg, qvn.reshape(_QT, hw2, _C),
                             (((2,), (2,)), ((0,), (0,))),
                             preferred_element_type=jnp.float32)
    s2v = s2.reshape(_QT, _N, hw2, hw2)
    irow = jax.lax.broadcasted_iota(jnp.int32, (_QT, _N, hw2, hw2), 2)
    topv = jnp.max(jnp.where(irow < _HW, s2v, _NEG), axis=2)
    jcol = jax.lax.broadcasted_iota(jnp.int32, (_QT, _N, hw2), 2)
    out_ref[0, 0] = jnp.sum(
        jnp.where(jcol < _HW, (topv + 1.0) * 0.5, 0.0), axis=2)


def _loss_body(sim_ref, y_ref, out_ref):
    s = sim_ref[...] * (1.0 / _TEMP)
    m = jnp.max(s, axis=1, keepdims=True)
    lse = m + jnp.log(jnp.sum(jnp.exp(s - m), axis=1, keepdims=True))
    logp = s - lse
    iota = jax.lax.broadcasted_iota(jnp.int32, s.shape, 1)
    picked = jnp.sum(jnp.where(iota == y_ref[...], logp, 0.0),
                     axis=1, keepdims=True)
    out_ref[...] = jnp.sum(-picked / picked.shape[0],
                           axis=0, keepdims=True)


def kernel(support_xf, support_y, query_xf, query_y, unlabeled_xf,
           Wk, Wq, Wv):
    b = support_xf.shape[0]
    q = query_xf.shape[1]
    f32 = jnp.float32

    sup_mc = (support_xf.reshape(b, _N, _K, _C, _HW)
              .transpose(0, 1, 3, 2, 4)
              .reshape(b, _N, _C, _MS)
              .transpose(0, 1, 3, 2)
              .reshape(b, _MST, _C))
    unl_mc = (unlabeled_xf.reshape(b, -1, _C, _HW)
              .transpose(0, 2, 1, 3)
              .reshape(b, _C, _MU)
              .transpose(0, 2, 1))
    qx5 = query_xf.reshape(b, _NQT, _QT, _C, _HW)

    skt, svt, ukt, vall, vunl, cnt = pl.pallas_call(
        _select_body,
        grid=(b,),
        in_specs=[
            pl.BlockSpec((1, _MST, _C), lambda i: (i, 0, 0)),
            pl.BlockSpec((1, _MU, _C), lambda i: (i, 0, 0)),
            pl.BlockSpec((_C, _C), lambda i: (0, 0)),
            pl.BlockSpec((_C, _C), lambda i: (0, 0)),
        ],
        out_specs=[
            pl.BlockSpec((1, _N, _MS, _C), lambda i: (i, 0, 0, 0)),
            pl.BlockSpec((1, _N, _MS, _C), lambda i: (i, 0, 0, 0)),
            pl.BlockSpec((1, _MU, _C), lambda i: (i, 0, 0)),
            pl.BlockSpec((1, _MU, _VW), lambda i: (i, 0, 0)),
            pl.BlockSpec((1, 1, _MU), lambda i: (i, 0, 0)),
            pl.BlockSpec((1, 1, _N), lambda i: (i, 0, 0)),
        ],
        out_shape=[
            jax.ShapeDtypeStruct((b, _N, _MS, _C), f32),
            jax.ShapeDtypeStruct((b, _N, _MS, _C), f32),
            jax.ShapeDtypeStruct((b, _MU, _C), f32),
            jax.ShapeDtypeStruct((b, _MU, _VW), f32),
            jax.ShapeDtypeStruct((b, 1, _MU), f32),
            jax.ShapeDtypeStruct((b, 1, _N), f32),
        ],
        compiler_params=pltpu.CompilerParams(
            dimension_semantics=("parallel",)),
    )(sup_mc, unl_mc, Wk, Wv)

    sim = pl.pallas_call(
        _main_body,
        grid=(b, _NQT),
        in_specs=[
            pl.BlockSpec((1, _N, _MS, _C), lambda i, j: (i, 0, 0, 0)),
            pl.BlockSpec((1, _N, _MS, _C), lambda i, j: (i, 0, 0, 0)),
            pl.BlockSpec((1, _MU, _C), lambda i, j: (i, 0, 0)),
            pl.BlockSpec((1, _MU, _VW), lambda i, j: (i, 0, 0)),
            pl.BlockSpec((1, 1, _QT, _C, _HW), lambda i, j: (i, j, 0, 0, 0)),
            pl.BlockSpec((_C, _C), lambda i, j: (0, 0)),
            pl.BlockSpec((_C, _C), lambda i, j: (0, 0)),
            pl.BlockSpec((1, 1, _MU), lambda i, j: (i, 0, 0)),
            pl.BlockSpec((b, 1, _N), lambda i, j: (0, 0, 0)),
        ],
        out_specs=pl.BlockSpec((1, 1, _QT, _N), lambda i, j: (i, j, 0, 0)),
        out_shape=jax.ShapeDtypeStruct((b, _NQT, _QT, _N), f32),
        compiler_params=pltpu.CompilerParams(
            dimension_semantics=("parallel", "arbitrary")),
    )(skt, svt, ukt, vall, qx5, Wq, Wv, vunl, cnt)

    loss = pl.pallas_call(
        _loss_body,
        grid=(1,),
        in_specs=[
            pl.BlockSpec((b * q, _N), lambda i: (0, 0)),
            pl.BlockSpec((b * q, 1), lambda i: (0, 0)),
        ],
        out_specs=pl.BlockSpec((1, 1), lambda i: (0, 0)),
        out_shape=jax.ShapeDtypeStruct((1, 1), f32),
    )(sim.reshape(b * q, _N), query_y.reshape(b * q, 1).astype(jnp.int32))

    return loss.reshape(())


# speedup vs baseline: 1.6868x; 1.0274x over previous
import jax
import jax.numpy as jnp
from jax.experimental import pallas as pl
from jax.experimental.pallas import tpu as pltpu

_N = 5
_K = 5
_C = 64
_HW = 25
_MS = _K * _HW
_MST = _N * _MS
_MU = 2500
_QT = 25
_NQT = 3
_INVSQ = 0.125
_TEMP = 2.0
_NEG = -jnp.inf
_VW = _N * _C + _N


def _dot(a, b, dims):
    return jax.lax.dot_general(a, b, (dims, ((), ())),
                               preferred_element_type=jnp.float32)


def _select_body(sup_ref, unl_ref, wk_ref, wv_ref,
                 skt_ref, svt_ref, ukt_ref, vall_ref, vunl_ref, cnt_ref):
    sup = sup_ref[0]
    unl = unl_ref[0]
    wk = wk_ref[...]
    wv = wv_ref[...]
    cn = ((1,), (1,))
    skt_ref[0] = _dot(sup, wk, cn).reshape(_N, _MS, _C)
    svt_ref[0] = _dot(sup, wv, cn).reshape(_N, _MS, _C)
    ukt_ref[0] = _dot(unl, wk, cn)
    uvt = _dot(unl, wv, cn)

    sn = sup / jnp.maximum(
        jnp.sqrt(jnp.sum(sup * sup, axis=1, keepdims=True)), 1e-12)
    sn3 = sn.reshape(_N, _MS, _C)
    un = unl / jnp.maximum(
        jnp.sqrt(jnp.sum(unl * unl, axis=1, keepdims=True)), 1e-12)

    ch = 2500
    nch = _MU // ch
    rowmaxs, unears, umax5s = [], [], []
    colmax = [jnp.full((1, _MS), _NEG, jnp.float32) for _ in range(_N)]
    iota = jax.lax.broadcasted_iota(jnp.int32, (ch, _MS), 1)
    for ci in range(nch):
        uc = un[ci * ch:(ci + 1) * ch]
        ccs = [_dot(uc, sn3[n], cn) for n in range(_N)]
        umax5 = jnp.concatenate(
            [jnp.max(ccs[n], axis=1, keepdims=True) for n in range(_N)],
            axis=1)
        rm = jnp.max(umax5, axis=1, keepdims=True)
        unear = jnp.full((ch, 1), _MST, jnp.int32)
        for n in range(_N):
            cand = jnp.min(jnp.where(ccs[n] >= rm, iota + n * _MS, _MST),
                           axis=1, keepdims=True)
            unear = jnp.minimum(unear, cand)
            colmax[n] = jnp.maximum(colmax[n],
                                    jnp.max(ccs[n], axis=0, keepdims=True))
        rowmaxs.append(rm)
        unears.append(unear)
        umax5s.append(umax5)

    mnns = []
    for ci in range(nch):
        cg = jnp.full((ch, 1), _NEG, jnp.float32)
        for n in range(_N):
            hit = jnp.where(unears[ci] == iota + n * _MS, colmax[n], _NEG)
            cg = jnp.maximum(cg, jnp.max(hit, axis=1, keepdims=True))
        mnns.append((rowmaxs[ci] >= cg).astype(jnp.float32))
    mnn = jnp.concatenate(mnns, axis=0)
    umax5 = jnp.concatenate(umax5s, axis=0)

    best = jnp.max(umax5, axis=1, keepdims=True)
    taken = jnp.zeros((_MU, 1), jnp.float32)
    rows = []
    for n in range(_N):
        sel = jnp.where(umax5[:, n:n + 1] >= best, 1.0 - taken, 0.0)
        taken = taken + sel
        rows.append(sel * mnn)
    um = jnp.concatenate(rows, axis=1)

    vall_ref[0] = jnp.concatenate(
        [rows[n] * uvt for n in range(_N)] + [um], axis=1)
    vunl_ref[0] = mnn.T
    cnt_ref[0] = jnp.sum(um, axis=0, keepdims=True)


def _main_body(skt_ref, svt_ref, ukt_ref, vall_ref, qx_ref, wq_ref, wv_ref,
               vunl_ref, cnt_ref, out_ref):
    ib = pl.program_id(0)
    skt = skt_ref[0]
    svt = svt_ref[0]
    ukt = ukt_ref[0]
    vall = vall_ref[0]
    qx = qx_ref[0, 0]
    wq = wq_ref[...]
    wv = wv_ref[...]

    l_sel = jnp.max(cnt_ref[...])
    cnt_b = cnt_ref[pl.ds(ib, 1)].reshape(1, _N)

    hw2 = 32
    qr = _QT * hw2
    qxt = jnp.concatenate(
        [jnp.transpose(qx, (0, 2, 1)),
         jnp.zeros((_QT, hw2 - _HW, _C), jnp.float32)], axis=1
    ).reshape(qr, _C)
    qk = _dot(qxt, wq, ((1,), (1,)))
    qv = _dot(qxt, wv, ((1,), (1,)))
    qvn = qv / jnp.maximum(
        jnp.sqrt(jnp.sum(qv * qv, axis=1, keepdims=True)), 1e-12)

    ls = [_dot(qk, skt[n], ((1,), (1,))) * _INVSQ for n in range(_N)]
    lu = _dot(qk, ukt, ((1,), (1,))) * _INVSQ

    vu = vunl_ref[0] > 0.0
    rs = jnp.max(ls[0], axis=1, keepdims=True)
    for n in range(1, _N):
        rs = jnp.maximum(rs, jnp.max(ls[n], axis=1, keepdims=True))
    ru = jnp.max(jnp.where(vu, lu, _NEG), axis=1, keepdims=True)
    rmax = jnp.maximum(rs, ru)
    rmax3 = rmax.reshape(_QT, hw2, 1)
    lu3 = lu.reshape(_QT, hw2, _MU)
    cu = jnp.max(lu3, axis=1, keepdims=True)
    mut = jnp.max(jnp.where((lu3 >= rmax3) & (lu3 >= cu) & vu[None],
                            1.0, 0.0), axis=2)
    for n in range(_N):
        ls3 = ls[n].reshape(_QT, hw2, _MS)
        cs = jnp.max(ls3, axis=1, keepdims=True)
        mut = jnp.maximum(mut, jnp.max(
            jnp.where((ls3 >= rmax3) & (ls3 >= cs), 1.0, 0.0), axis=2))
    iota2 = jax.lax.broadcasted_iota(jnp.int32, (_QT, hw2), 1)
    first = jnp.where(iota2 == 0, 1.0, 0.0)
    rowvalid = jnp.where(iota2 < _HW, 1.0, 0.0)
    qm = jnp.where(rmax3[:, :, 0] >= 0.0, mut, first) * rowvalid
    qmf = qm.reshape(qr, 1)

    lum = lu * qmf
    g = jnp.max(lum, axis=1, keepdims=True)
    lsm = []
    for n in range(_N):
        lsmn = ls[n] * qmf
        lsm.append(lsmn)
        g = jnp.maximum(g, jnp.max(lsmn, axis=1, keepdims=True))
    mp = jnp.maximum(g, 0.0)
    emn = jnp.exp(-mp)
    punl = jnp.exp(lum - mp)
    unl_out = _dot(punl, vall, ((1,), (0,)))

    aligned = []
    for n in range(_N):
        psup = jnp.exp(lsm[n] - mp)
        val = (_dot(psup, svt[n], ((1,), (0,)))
               + unl_out[:, n * _C:(n + 1) * _C])
        padc = l_sel - cnt_b[0, n]
        den = (jnp.sum(psup, axis=1, keepdims=True)
               + unl_out[:, _N * _C + n:_N * _C + n + 1]
               + padc * emn)
        al = val / den
        al = al / jnp.maximum(
            jnp.sqrt(jnp.sum(al * al, axis=1, keepdims=True)), 1e-12)
        aligned.append(al.reshape(_QT, 1, hw2, _C))
    alg = jnp.concatenate(aligned, axis=1).reshape(_QT, _N * hw2, _C)

    s2 = jax.lax.dot_general(alg, qvn.reshape(_QT, hw2, _C),
                             (((2,), (2,)), ((0,), (0,))),
                             preferred_element_type=jnp.float32)
    s2v = s2.reshape(_QT, _N, hw2, hw2)
    irow = jax.lax.broadcasted_iota(jnp.int32, (_QT, _N, hw2, hw2), 2)
    topv = jnp.max(jnp.where(irow < _HW, s2v, _NEG), axis=2)
    jcol = jax.lax.broadcasted_iota(jnp.int32, (_QT, _N, hw2), 2)
    out_ref[0, 0] = jnp.sum(
        jnp.where(jcol < _HW, (topv + 1.0) * 0.5, 0.0), axis=2)


def _loss_body(sim_ref, y_ref, out_ref):
    s = sim_ref[...] * (1.0 / _TEMP)
    m = jnp.max(s, axis=1, keepdims=True)
    lse = m + jnp.log(jnp.sum(jnp.exp(s - m), axis=1, keepdims=True))
    logp = s - lse
    iota = jax.lax.broadcasted_iota(jnp.int32, s.shape, 1)
    picked = jnp.sum(jnp.where(iota == y_ref[...], logp, 0.0),
                     axis=1, keepdims=True)
    out_ref[...] = jnp.sum(-picked / picked.shape[0],
                           axis=0, keepdims=True)


def kernel(support_xf, support_y, query_xf, query_y, unlabeled_xf,
           Wk, Wq, Wv):
    b = support_xf.shape[0]
    q = query_xf.shape[1]
    f32 = jnp.float32

    sup_mc = (support_xf.reshape(b, _N, _K, _C, _HW)
              .transpose(0, 1, 3, 2, 4)
              .reshape(b, _N, _C, _MS)
              .transpose(0, 1, 3, 2)
              .reshape(b, _MST, _C))
    unl_mc = (unlabeled_xf.reshape(b, -1, _C, _HW)
              .transpose(0, 2, 1, 3)
              .reshape(b, _C, _MU)
              .transpose(0, 2, 1))
    qx5 = query_xf.reshape(b, _NQT, _QT, _C, _HW)

    skt, svt, ukt, vall, vunl, cnt = pl.pallas_call(
        _select_body,
        grid=(b,),
        in_specs=[
            pl.BlockSpec((1, _MST, _C), lambda i: (i, 0, 0)),
            pl.BlockSpec((1, _MU, _C), lambda i: (i, 0, 0)),
            pl.BlockSpec((_C, _C), lambda i: (0, 0)),
            pl.BlockSpec((_C, _C), lambda i: (0, 0)),
        ],
        out_specs=[
            pl.BlockSpec((1, _N, _MS, _C), lambda i: (i, 0, 0, 0)),
            pl.BlockSpec((1, _N, _MS, _C), lambda i: (i, 0, 0, 0)),
            pl.BlockSpec((1, _MU, _C), lambda i: (i, 0, 0)),
            pl.BlockSpec((1, _MU, _VW), lambda i: (i, 0, 0)),
            pl.BlockSpec((1, 1, _MU), lambda i: (i, 0, 0)),
            pl.BlockSpec((1, 1, _N), lambda i: (i, 0, 0)),
        ],
        out_shape=[
            jax.ShapeDtypeStruct((b, _N, _MS, _C), f32),
            jax.ShapeDtypeStruct((b, _N, _MS, _C), f32),
            jax.ShapeDtypeStruct((b, _MU, _C), f32),
            jax.ShapeDtypeStruct((b, _MU, _VW), f32),
            jax.ShapeDtypeStruct((b, 1, _MU), f32),
            jax.ShapeDtypeStruct((b, 1, _N), f32),
        ],
        compiler_params=pltpu.CompilerParams(
            dimension_semantics=("parallel",)),
    )(sup_mc, unl_mc, Wk, Wv)

    sim = pl.pallas_call(
        _main_body,
        grid=(b, _NQT),
        in_specs=[
            pl.BlockSpec((1, _N, _MS, _C), lambda i, j: (i, 0, 0, 0)),
            pl.BlockSpec((1, _N, _MS, _C), lambda i, j: (i, 0, 0, 0)),
            pl.BlockSpec((1, _MU, _C), lambda i, j: (i, 0, 0)),
            pl.BlockSpec((1, _MU, _VW), lambda i, j: (i, 0, 0)),
            pl.BlockSpec((1, 1, _QT, _C, _HW), lambda i, j: (i, j, 0, 0, 0)),
            pl.BlockSpec((_C, _C), lambda i, j: (0, 0)),
            pl.BlockSpec((_C, _C), lambda i, j: (0, 0)),
            pl.BlockSpec((1, 1, _MU), lambda i, j: (i, 0, 0)),
            pl.BlockSpec((b, 1, _N), lambda i, j: (0, 0, 0)),
        ],
        out_specs=pl.BlockSpec((1, 1, _QT, _N), lambda i, j: (i, j, 0, 0)),
        out_shape=jax.ShapeDtypeStruct((b, _NQT, _QT, _N), f32),
        compiler_params=pltpu.CompilerParams(
            dimension_semantics=("parallel", "arbitrary")),
    )(skt, svt, ukt, vall, qx5, Wq, Wv, vunl, cnt)

    loss = pl.pallas_call(
        _loss_body,
        grid=(1,),
        in_specs=[
            pl.BlockSpec((b * q, _N), lambda i: (0, 0)),
            pl.BlockSpec((b * q, 1), lambda i: (0, 0)),
        ],
        out_specs=pl.BlockSpec((1, 1), lambda i: (0, 0)),
        out_shape=jax.ShapeDtypeStruct((1, 1), f32),
    )(sim.reshape(b * q, _N), query_y.reshape(b * q, 1).astype(jnp.int32))

    return loss.reshape(())
